# baseline (speedup 1.0000x reference)
import jax
import jax.numpy as jnp
from jax import lax
from jax.experimental import pallas as pl
from jax.experimental.pallas import tpu as pltpu

N = 2048
H = 128
BM = 512
G = N // BM
SIM_T = 0.9
_DN = (((1,), (1,)), ((), ()))
_AH = (((1,), (0,)), ((), ()))


def _fused_body(h0_ref, satc_ref, satr_ref, wl0_ref, bl0_ref, wr0_ref,
                wl1_ref, bl1_ref, wr1_ref, g_ref, b_ref, wo_ref, bo_ref,
                hbn_ref, out_ref,
                fn_ref, hhi_ref, hlo_ref, am_ref, deg_ref, h1_ref, h2_ref):
    s = pl.program_id(0)

    def split_h(src_ref):
        h = src_ref[...]
        hhi = h.astype(jnp.bfloat16)
        hhi_ref[...] = hhi
        hlo_ref[...] = (h - hhi.astype(jnp.float32)).astype(jnp.bfloat16)

    def sage_block(i, hsrc_ref, wl_ref, bl_ref, wr_ref, dst_ref):
        a = am_ref[pl.ds(i * BM, BM), :]
        deg = deg_ref[pl.ds(i * BM, BM), :]
        agg = (lax.dot_general(a, hhi_ref[...], _AH,
                               preferred_element_type=jnp.float32)
               + lax.dot_general(a, hlo_ref[...], _AH,
                                 preferred_element_type=jnp.float32))
        hblk = hsrc_ref[pl.ds(i * BM, BM), :]
        agg = (agg - hblk) / jnp.maximum(deg, 1.0)
        z = (lax.dot_general(agg, wl_ref[...], _DN)
             + bl_ref[...]
             + lax.dot_general(hblk, wr_ref[...], _DN))
        dst_ref[pl.ds(i * BM, BM), :] = jnp.maximum(z, 0.0)

    def build_mask(i):
        fblk = fn_ref[pl.ds(i * BM, BM), :]
        cos = lax.dot_general(fblk, fn_ref[...], _DN)
        satc = satc_ref[pl.ds(i * BM, BM), :]
        keep = (satc == satr_ref[...]) | (cos > SIM_T)
        af = jnp.where(keep, 1.0, 0.0)
        am_ref[pl.ds(i * BM, BM), :] = af.astype(jnp.bfloat16)
        deg_ref[pl.ds(i * BM, BM), :] = (
            jnp.sum(af, axis=1, keepdims=True) - 1.0)

    @pl.when(s == 0)
    def _prologue():
        x0 = h0_ref[...]
        lane = lax.broadcasted_iota(jnp.int32, x0.shape, 1)
        xm = jnp.where(lane < 4, x0, 0.0)
        ns = jnp.sum(xm * xm, axis=1, keepdims=True)
        fn_ref[...] = xm / jnp.maximum(jnp.sqrt(ns), 1e-12)
        split_h(h0_ref)
        build_mask(0)

    @pl.when(s < G)
    def _layer0():
        @pl.when(s + 1 < G)
        def _build_next():
            build_mask(s + 1)
        sage_block(s, h0_ref, wl0_ref, bl0_ref, wr0_ref, h1_ref)

    @pl.when(s == G)
    def _relsplit():
        split_h(h1_ref)

    @pl.when((s >= G) & (s < 2 * G))
    def _layer1():
        sage_block(s - G, h1_ref, wl1_ref, bl1_ref, wr1_ref, h2_ref)

    @pl.when(s == 2 * G)
    def _bn_out():
        h = h2_ref[...]
        mean = jnp.mean(h, axis=0, keepdims=True)
        var = jnp.mean((h - mean) ** 2, axis=0, keepdims=True)
        hbn = (h - mean) / jnp.sqrt(var + 1e-5) * g_ref[...] + b_ref[...]
        hbn_ref[...] = hbn
        out_ref[...] = lax.dot_general(hbn, wo_ref[...], _DN) + bo_ref[...]


_full = lambda shape: pl.BlockSpec(shape, lambda i: (0,) * len(shape))


def kernel(x_now, sat_type, sage0_Wl, sage0_bl, sage0_Wr, sage1_Wl, sage1_bl,
           sage1_Wr, bn_gamma, bn_beta, W_out, b_out):
    h0 = jnp.concatenate([x_now[:, :1], x_now], axis=1)
    sat = sat_type.astype(jnp.int32)
    w_pad = jnp.zeros((H, H), jnp.float32).at[:W_out.shape[0]].set(W_out)
    b_pad = jnp.zeros((1, H), jnp.float32).at[0, :b_out.shape[0]].set(b_out)

    hbn, out_pad = pl.pallas_call(
        _fused_body,
        grid=(2 * G + 1,),
        in_specs=[_full((N, H)), _full((N, 1)), _full((1, N)),
                  _full((H, H)), _full((1, H)), _full((H, H)),
                  _full((H, H)), _full((1, H)), _full((H, H)),
                  _full((1, H)), _full((1, H)), _full((H, H)),
                  _full((1, H))],
        out_specs=[pl.BlockSpec((N, H), lambda i: (0, 0)),
                   pl.BlockSpec((N, H), lambda i: (0, 0))],
        out_shape=[jax.ShapeDtypeStruct((N, H), jnp.float32),
                   jax.ShapeDtypeStruct((N, H), jnp.float32)],
        scratch_shapes=[pltpu.VMEM((N, H), jnp.float32),
                        pltpu.VMEM((N, H), jnp.bfloat16),
                        pltpu.VMEM((N, H), jnp.bfloat16),
                        pltpu.VMEM((N, N), jnp.bfloat16),
                        pltpu.VMEM((N, 1), jnp.float32),
                        pltpu.VMEM((N, H), jnp.float32),
                        pltpu.VMEM((N, H), jnp.float32)],
    )(h0, sat.reshape(N, 1), sat.reshape(1, N),
      sage0_Wl, sage0_bl.reshape(1, H), sage0_Wr,
      sage1_Wl, sage1_bl.reshape(1, H), sage1_Wr,
      bn_gamma.reshape(1, H), bn_beta.reshape(1, H), w_pad, b_pad)
    return (hbn, out_pad[:, :W_out.shape[0]])


# speedup vs baseline: 1.0615x; 1.0615x over previous
import jax
import jax.numpy as jnp
from jax import lax
from jax.experimental import pallas as pl
from jax.experimental.pallas import tpu as pltpu

N = 2048
H = 128
BM = 1024
G = N // BM
SIM_T = 0.9
_DN = (((1,), (1,)), ((), ()))
_AH = (((1,), (0,)), ((), ()))


def _fused_body(h0_ref, satc_ref, satr_ref, wl0_ref, bl0_ref, wr0_ref,
                wl1_ref, bl1_ref, wr1_ref, g_ref, b_ref, wo_ref, bo_ref,
                hbn_ref, out_ref,
                fn_ref, hhi_ref, hlo_ref, am_ref, deg_ref, h1_ref, h2_ref):
    s = pl.program_id(0)

    def split_h(src_ref):
        h = src_ref[...]
        hhi = h.astype(jnp.bfloat16)
        hhi_ref[...] = hhi
        hlo_ref[...] = (h - hhi.astype(jnp.float32)).astype(jnp.bfloat16)

    def sage_block(i, hsrc_ref, wl_ref, bl_ref, wr_ref, dst_ref):
        a = am_ref[pl.ds(i * BM, BM), :]
        deg = deg_ref[pl.ds(i * BM, BM), :]
        agg = (lax.dot_general(a, hhi_ref[...], _AH,
                               preferred_element_type=jnp.float32)
               + lax.dot_general(a, hlo_ref[...], _AH,
                                 preferred_element_type=jnp.float32))
        hblk = hsrc_ref[pl.ds(i * BM, BM), :]
        agg = (agg - hblk) / jnp.maximum(deg, 1.0)
        z = (lax.dot_general(agg, wl_ref[...], _DN)
             + bl_ref[...]
             + lax.dot_general(hblk, wr_ref[...], _DN))
        dst_ref[pl.ds(i * BM, BM), :] = jnp.maximum(z, 0.0)

    def build_mask(i):
        fblk = fn_ref[pl.ds(i * BM, BM), :]
        cos = lax.dot_general(fblk, fn_ref[...], _DN)
        satc = satc_ref[pl.ds(i * BM, BM), :]
        keep = (satc == satr_ref[...]) | (cos > SIM_T)
        af = jnp.where(keep, 1.0, 0.0)
        am_ref[pl.ds(i * BM, BM), :] = af.astype(jnp.bfloat16)
        deg_ref[pl.ds(i * BM, BM), :] = (
            jnp.sum(af, axis=1, keepdims=True) - 1.0)

    @pl.when(s == 0)
    def _prologue():
        x0 = h0_ref[...]
        lane = lax.broadcasted_iota(jnp.int32, x0.shape, 1)
        xm = jnp.where(lane < 4, x0, 0.0)
        ns = jnp.sum(xm * xm, axis=1, keepdims=True)
        fn_ref[...] = xm / jnp.maximum(jnp.sqrt(ns), 1e-12)
        split_h(h0_ref)
        build_mask(0)

    @pl.when(s < G)
    def _layer0():
        @pl.when(s + 1 < G)
        def _build_next():
            build_mask(s + 1)
        sage_block(s, h0_ref, wl0_ref, bl0_ref, wr0_ref, h1_ref)

    @pl.when(s == G)
    def _relsplit():
        split_h(h1_ref)

    @pl.when((s >= G) & (s < 2 * G))
    def _layer1():
        sage_block(s - G, h1_ref, wl1_ref, bl1_ref, wr1_ref, h2_ref)

    @pl.when(s == 2 * G)
    def _bn_out():
        h = h2_ref[...]
        mean = jnp.mean(h, axis=0, keepdims=True)
        var = jnp.mean((h - mean) ** 2, axis=0, keepdims=True)
        hbn = (h - mean) / jnp.sqrt(var + 1e-5) * g_ref[...] + b_ref[...]
        hbn_ref[...] = hbn
        out_ref[...] = lax.dot_general(hbn, wo_ref[...], _DN) + bo_ref[...]


_full = lambda shape: pl.BlockSpec(shape, lambda i: (0,) * len(shape))


def kernel(x_now, sat_type, sage0_Wl, sage0_bl, sage0_Wr, sage1_Wl, sage1_bl,
           sage1_Wr, bn_gamma, bn_beta, W_out, b_out):
    h0 = jnp.concatenate([x_now[:, :1], x_now], axis=1)
    sat = sat_type.astype(jnp.int32)
    w_pad = jnp.zeros((H, H), jnp.float32).at[:W_out.shape[0]].set(W_out)
    b_pad = jnp.zeros((1, H), jnp.float32).at[0, :b_out.shape[0]].set(b_out)

    hbn, out_pad = pl.pallas_call(
        _fused_body,
        grid=(2 * G + 1,),
        in_specs=[_full((N, H)), _full((N, 1)), _full((1, N)),
                  _full((H, H)), _full((1, H)), _full((H, H)),
                  _full((H, H)), _full((1, H)), _full((H, H)),
                  _full((1, H)), _full((1, H)), _full((H, H)),
                  _full((1, H))],
        out_specs=[pl.BlockSpec((N, H), lambda i: (0, 0)),
                   pl.BlockSpec((N, H), lambda i: (0, 0))],
        out_shape=[jax.ShapeDtypeStruct((N, H), jnp.float32),
                   jax.ShapeDtypeStruct((N, H), jnp.float32)],
        scratch_shapes=[pltpu.VMEM((N, H), jnp.float32),
                        pltpu.VMEM((N, H), jnp.bfloat16),
                        pltpu.VMEM((N, H), jnp.bfloat16),
                        pltpu.VMEM((N, N), jnp.bfloat16),
                        pltpu.VMEM((N, 1), jnp.float32),
                        pltpu.VMEM((N, H), jnp.float32),
                        pltpu.VMEM((N, H), jnp.float32)],
    )(h0, sat.reshape(N, 1), sat.reshape(1, N),
      sage0_Wl, sage0_bl.reshape(1, H), sage0_Wr,
      sage1_Wl, sage1_bl.reshape(1, H), sage1_Wr,
      bn_gamma.reshape(1, H), bn_beta.reshape(1, H), w_pad, b_pad)
    return (hbn, out_pad[:, :W_out.shape[0]])


# speedup vs baseline: 1.2269x; 1.1559x over previous
import jax
import jax.numpy as jnp
from jax import lax
from jax.experimental import pallas as pl
from jax.experimental.pallas import tpu as pltpu

N = 2048
H = 128
BM = 1024
G = N // BM
SIM_T = 0.9
_DN = (((1,), (1,)), ((), ()))
_AH = (((1,), (0,)), ((), ()))


def _fused_body(h0_ref, satc_ref, satr_ref, wl0_ref, bl0_ref, wr0_ref,
                wl1_ref, bl1_ref, wr1_ref, g_ref, b_ref, wo_ref, bo_ref,
                hbn_ref, out_ref,
                fn_ref, hhi_ref, hlo_ref, am_ref, deg_ref, h1_ref, h2_ref):
    s = pl.program_id(0)

    def split_h(src_ref):
        h = src_ref[...]
        hhi = h.astype(jnp.bfloat16)
        hhi_ref[...] = hhi
        hlo_ref[...] = (h - hhi.astype(jnp.float32)).astype(jnp.bfloat16)

    def sage_block(i, hsrc_ref, wl_ref, bl_ref, wr_ref, dst_ref):
        a = am_ref[pl.ds(i * BM, BM), :]
        deg = deg_ref[pl.ds(i * BM, BM), :]
        agg = (lax.dot_general(a, hhi_ref[...], _AH,
                               preferred_element_type=jnp.float32)
               + lax.dot_general(a, hlo_ref[...], _AH,
                                 preferred_element_type=jnp.float32))
        hblk = hsrc_ref[pl.ds(i * BM, BM), :]
        agg = (agg - hblk) / jnp.maximum(deg, 1.0)
        z = (lax.dot_general(agg, wl_ref[...], _DN)
             + bl_ref[...]
             + lax.dot_general(hblk, wr_ref[...], _DN))
        dst_ref[pl.ds(i * BM, BM), :] = jnp.maximum(z, 0.0)

    def build_mask(i):
        fblk = fn_ref[pl.ds(i * BM, BM), :]
        cos = lax.dot_general(fblk, fn_ref[...], _DN)
        satc = satc_ref[pl.ds(i * BM, BM), :]
        keep = (satc == satr_ref[...]) | (cos > SIM_T)
        af = jnp.where(keep, 1.0, 0.0)
        am_ref[pl.ds(i * BM, BM), :] = af.astype(jnp.bfloat16)
        deg_ref[pl.ds(i * BM, BM), :] = (
            jnp.sum(af, axis=1, keepdims=True) - 1.0)

    @pl.when(s == 0)
    def _prologue():
        x0 = h0_ref[...]
        lane = lax.broadcasted_iota(jnp.int32, x0.shape, 1)
        xm = jnp.where(lane < 4, x0, 0.0)
        ns = jnp.sum(xm * xm, axis=1, keepdims=True)
        fn_ref[...] = xm / jnp.maximum(jnp.sqrt(ns), 1e-12)
        split_h(h0_ref)
        build_mask(0)

    @pl.when(s < G)
    def _layer0():
        @pl.when(s + 1 < G)
        def _build_next():
            build_mask(s + 1)
        sage_block(s, h0_ref, wl0_ref, bl0_ref, wr0_ref, h1_ref)

    @pl.when(s == G)
    def _relsplit():
        split_h(h1_ref)

    @pl.when(s >= G)
    def _layer1():
        sage_block(s - G, h1_ref, wl1_ref, bl1_ref, wr1_ref, h2_ref)

    @pl.when(s == 2 * G - 1)
    def _bn_out():
        h = h2_ref[...]
        mean = jnp.mean(h, axis=0, keepdims=True)
        var = jnp.mean((h - mean) ** 2, axis=0, keepdims=True)
        hbn = (h - mean) / jnp.sqrt(var + 1e-5) * g_ref[...] + b_ref[...]
        hbn_ref[...] = hbn
        out_ref[...] = lax.dot_general(hbn, wo_ref[...], _DN) + bo_ref[...]


_full = lambda shape: pl.BlockSpec(shape, lambda i: (0,) * len(shape))


def kernel(x_now, sat_type, sage0_Wl, sage0_bl, sage0_Wr, sage1_Wl, sage1_bl,
           sage1_Wr, bn_gamma, bn_beta, W_out, b_out):
    h0 = jnp.concatenate([x_now[:, :1], x_now], axis=1)
    sat = sat_type.astype(jnp.int32)
    nout = W_out.shape[0]

    hbn, out = pl.pallas_call(
        _fused_body,
        grid=(2 * G,),
        in_specs=[_full((N, H)), _full((N, 1)), _full((1, N)),
                  _full((H, H)), _full((1, H)), _full((H, H)),
                  _full((H, H)), _full((1, H)), _full((H, H)),
                  _full((1, H)), _full((1, H)), _full((nout, H)),
                  _full((1, nout))],
        out_specs=[pl.BlockSpec((N, H), lambda i: (0, 0)),
                   pl.BlockSpec((N, nout), lambda i: (0, 0))],
        out_shape=[jax.ShapeDtypeStruct((N, H), jnp.float32),
                   jax.ShapeDtypeStruct((N, nout), jnp.float32)],
        scratch_shapes=[pltpu.VMEM((N, H), jnp.float32),
                        pltpu.VMEM((N, H), jnp.bfloat16),
                        pltpu.VMEM((N, H), jnp.bfloat16),
                        pltpu.VMEM((N, N), jnp.bfloat16),
                        pltpu.VMEM((N, 1), jnp.float32),
                        pltpu.VMEM((N, H), jnp.float32),
                        pltpu.VMEM((N, H), jnp.float32)],
    )(h0, sat.reshape(N, 1), sat.reshape(1, N),
      sage0_Wl, sage0_bl.reshape(1, H), sage0_Wr,
      sage1_Wl, sage1_bl.reshape(1, H), sage1_Wr,
      bn_gamma.reshape(1, H), bn_beta.reshape(1, H), W_out,
      b_out.reshape(1, nout))
    return (hbn, out)


# speedup vs baseline: 1.2533x; 1.0215x over previous
import jax
import jax.numpy as jnp
from jax import lax
from jax.experimental import pallas as pl
from jax.experimental.pallas import tpu as pltpu

N = 2048
H = 128
BM = 1024
G = N // BM
SIM_T = 0.9
_DN = (((1,), (1,)), ((), ()))
_AH = (((1,), (0,)), ((), ()))


def _fused_body(x_ref, satc_ref, satr_ref, wl0_ref, bl0_ref, wr0_ref,
                wl1_ref, bl1_ref, wr1_ref, g_ref, b_ref, wo_ref, bo_ref,
                hbn_ref, out_ref,
                h0_ref, fn_ref, hhi_ref, hlo_ref, am_ref, deg_ref, h1_ref,
                h2_ref):
    s = pl.program_id(0)

    def split_h(src_ref):
        h = src_ref[...]
        hhi = h.astype(jnp.bfloat16)
        hhi_ref[...] = hhi
        hlo_ref[...] = (h - hhi.astype(jnp.float32)).astype(jnp.bfloat16)

    def sage_block(i, hsrc_ref, wl_ref, bl_ref, wr_ref, dst_ref):
        a = am_ref[pl.ds(i * BM, BM), :]
        deg = deg_ref[pl.ds(i * BM, BM), :]
        agg = (lax.dot_general(a, hhi_ref[...], _AH,
                               preferred_element_type=jnp.float32)
               + lax.dot_general(a, hlo_ref[...], _AH,
                                 preferred_element_type=jnp.float32))
        hblk = hsrc_ref[pl.ds(i * BM, BM), :]
        agg = (agg - hblk) / jnp.maximum(deg, 1.0)
        z = (lax.dot_general(agg, wl_ref[...], _DN)
             + bl_ref[...]
             + lax.dot_general(hblk, wr_ref[...], _DN))
        dst_ref[pl.ds(i * BM, BM), :] = jnp.maximum(z, 0.0)

    def build_mask(i):
        fblk = fn_ref[pl.ds(i * BM, BM), :]
        cos = lax.dot_general(fblk, fn_ref[...], _DN)
        satc = satc_ref[pl.ds(i * BM, BM), :]
        keep = (satc == satr_ref[...]) | (cos > SIM_T)
        af = jnp.where(keep, 1.0, 0.0)
        am_ref[pl.ds(i * BM, BM), :] = af.astype(jnp.bfloat16)
        deg_ref[pl.ds(i * BM, BM), :] = (
            jnp.sum(af, axis=1, keepdims=True) - 1.0)

    @pl.when(s == 0)
    def _prologue():
        x = x_ref[...]
        x0 = jnp.concatenate([x[:, :1], x], axis=1)
        h0_ref[...] = x0
        lane = lax.broadcasted_iota(jnp.int32, x0.shape, 1)
        xm = jnp.where(lane < 4, x0, 0.0)
        ns = jnp.sum(xm * xm, axis=1, keepdims=True)
        fn_ref[...] = xm / jnp.maximum(jnp.sqrt(ns), 1e-12)
        split_h(h0_ref)
        build_mask(0)

    @pl.when(s < G)
    def _layer0():
        @pl.when(s + 1 < G)
        def _build_next():
            build_mask(s + 1)
        sage_block(s, h0_ref, wl0_ref, bl0_ref, wr0_ref, h1_ref)

    @pl.when(s == G)
    def _relsplit():
        split_h(h1_ref)

    @pl.when(s >= G)
    def _layer1():
        sage_block(s - G, h1_ref, wl1_ref, bl1_ref, wr1_ref, h2_ref)

    @pl.when(s == 2 * G - 1)
    def _bn_out():
        h = h2_ref[...]
        mean = jnp.mean(h, axis=0, keepdims=True)
        var = jnp.mean((h - mean) ** 2, axis=0, keepdims=True)
        hbn = (h - mean) / jnp.sqrt(var + 1e-5) * g_ref[...] + b_ref[...]
        hbn_ref[...] = hbn
        out_ref[...] = lax.dot_general(hbn, wo_ref[...], _DN) + bo_ref[...]


_full = lambda shape: pl.BlockSpec(shape, lambda i: (0,) * len(shape))


def kernel(x_now, sat_type, sage0_Wl, sage0_bl, sage0_Wr, sage1_Wl, sage1_bl,
           sage1_Wr, bn_gamma, bn_beta, W_out, b_out):
    sat = sat_type.astype(jnp.int32)
    nout = W_out.shape[0]
    nd = x_now.shape[1]

    hbn, out = pl.pallas_call(
        _fused_body,
        grid=(2 * G,),
        in_specs=[_full((N, nd)), _full((N, 1)), _full((1, N)),
                  _full((H, H)), _full((1, H)), _full((H, H)),
                  _full((H, H)), _full((1, H)), _full((H, H)),
                  _full((1, H)), _full((1, H)), _full((nout, H)),
                  _full((1, nout))],
        out_specs=[pl.BlockSpec((N, H), lambda i: (0, 0)),
                   pl.BlockSpec((N, nout), lambda i: (0, 0))],
        out_shape=[jax.ShapeDtypeStruct((N, H), jnp.float32),
                   jax.ShapeDtypeStruct((N, nout), jnp.float32)],
        scratch_shapes=[pltpu.VMEM((N, H), jnp.float32),
                        pltpu.VMEM((N, H), jnp.float32),
                        pltpu.VMEM((N, H), jnp.bfloat16),
                        pltpu.VMEM((N, H), jnp.bfloat16),
                        pltpu.VMEM((N, N), jnp.bfloat16),
                        pltpu.VMEM((N, 1), jnp.float32),
                        pltpu.VMEM((N, H), jnp.float32),
                        pltpu.VMEM((N, H), jnp.float32)],
    )(x_now, sat.reshape(N, 1), sat.reshape(1, N),
      sage0_Wl, sage0_bl.reshape(1, H), sage0_Wr,
      sage1_Wl, sage1_bl.reshape(1, H), sage1_Wr,
      bn_gamma.reshape(1, H), bn_beta.reshape(1, H), W_out,
      b_out.reshape(1, nout))
    return (hbn, out)
